# baseline (speedup 1.0000x reference)
import functools

import jax
import jax.numpy as jnp
from jax.experimental import pallas as pl
from jax.experimental.pallas import tpu as pltpu

_R = 64
_NEG = -1e30


def _proj_kernel(idx_ref, f_ref, pv_ref, n_ref, wf_ref, wx01_ref, wxc_ref,
                 b_ref, o_ref, zbuf, a0, a1, a2, a3, a4, a5, a6, a7, *,
                 ncb, unroll, pb):
    b = pl.program_id(0)
    c = pl.program_id(1)
    nch = pl.num_programs(1)
    bufs = (a0, a1, a2, a3, a4, a5, a6, a7)

    @pl.when(c == 0)
    def _init():
        for buf in bufs:
            buf[0:2 * pb:2, :] = jnp.full((pb, 128), _NEG, jnp.float32)
            buf[1:2 * pb:2, :] = jnp.zeros((pb, 128), jnp.float32)

    norm = n_ref[...]
    xpyp = pv_ref[...][:, 2:4]
    z = (jnp.dot(f_ref[...], wf_ref[...],
                 preferred_element_type=jnp.float32)
         + jnp.dot(xpyp, wx01_ref[...],
                   preferred_element_type=jnp.float32)
         + jnp.dot(norm, wxc_ref[...],
                   preferred_element_type=jnp.float32)
         + b_ref[...])
    zbuf[0:2 * ncb:2, :] = z
    zbuf[1:2 * ncb:2, 0:4] = jnp.concatenate(
        [norm, jnp.ones((ncb, 1), jnp.float32)], axis=1)

    mask2 = jax.lax.broadcasted_iota(jnp.int32, (2, 128), 0) == 0
    coff = c * ncb

    def body(j, carry):
        k0 = j * unroll
        for g in range(unroll // 8):
            ks = [k0 + 8 * g + t for t in range(8)]
            i2s = [pl.multiple_of(idx_ref[b, coff + k], 2) for k in ks]
            zns = [zbuf[pl.ds(2 * k, 2), :] for k in ks]
            olds = [bufs[t][pl.ds(i2s[t], 2), :] for t in range(8)]
            news = [jnp.where(mask2, jnp.maximum(olds[t], zns[t]),
                              olds[t] + zns[t]) for t in range(8)]
            for t in range(8):
                bufs[t][pl.ds(i2s[t], 2), :] = news[t]
        return carry

    jax.lax.fori_loop(0, ncb // unroll, body, 0)

    @pl.when(c == nch - 1)
    def _fin():
        ms = [buf[0:2 * pb:2, :] for buf in bufs]
        while len(ms) > 1:
            ms = [jnp.maximum(ms[i], ms[i + 1]) for i in range(0, len(ms), 2)]
        m = ms[0]
        ss = [buf[1:2 * pb:2, 0:4] for buf in bufs]
        while len(ss) > 1:
            ss = [ss[i] + ss[i + 1] for i in range(0, len(ss), 2)]
        s = ss[0]
        cnt = jnp.maximum(s[:, 3:4], 1.0)
        mean = s[:, 0:3] / cnt
        corr = jnp.dot(mean, wxc_ref[...],
                       preferred_element_type=jnp.float32)
        o_ref[...] = jnp.maximum(m - corr, 0.0)


def kernel(features, norm_coords, coords_int, p_v_dist, wf, wx, b_eff):
    B, C, Np = features.shape
    N = B * Np
    Cout = wf.shape[1]
    R = _R
    PB = R * R

    points = jnp.transpose(features, (0, 2, 1)).reshape(N, C)
    wx01 = wx[0:2]
    wxc = wx[2:5]
    li2 = ((coords_int[:, 2] * R + coords_int[:, 3]) * 2).astype(
        jnp.int32).reshape(B, Np)

    NCB = 2048
    while Np % NCB:
        NCB //= 2
    nch = Np // NCB
    UNROLL = 32

    kb = functools.partial(_proj_kernel, ncb=NCB, unroll=UNROLL, pb=PB)
    out2 = pl.pallas_call(
        kb,
        out_shape=jax.ShapeDtypeStruct((B * PB, Cout), jnp.float32),
        grid=(B, nch),
        in_specs=[
            pl.BlockSpec(memory_space=pltpu.SMEM),
            pl.BlockSpec((NCB, C), lambda b, c: (b * (Np // NCB) + c, 0)),
            pl.BlockSpec((NCB, 4), lambda b, c: (b * (Np // NCB) + c, 0)),
            pl.BlockSpec((NCB, 3), lambda b, c: (b * (Np // NCB) + c, 0)),
            pl.BlockSpec((C, Cout), lambda b, c: (0, 0)),
            pl.BlockSpec((2, Cout), lambda b, c: (0, 0)),
            pl.BlockSpec((3, Cout), lambda b, c: (0, 0)),
            pl.BlockSpec((1, Cout), lambda b, c: (0, 0)),
        ],
        out_specs=pl.BlockSpec((PB, Cout), lambda b, c: (b, 0)),
        scratch_shapes=[pltpu.VMEM((2 * NCB, Cout), jnp.float32)] +
                       [pltpu.VMEM((2 * PB, Cout), jnp.float32)
                        for _ in range(8)],
        compiler_params=pltpu.CompilerParams(
            dimension_semantics=("parallel", "arbitrary"),
            vmem_limit_bytes=48 * 1024 * 1024,
        ),
    )(li2, points, p_v_dist, norm_coords, wf, wx01, wxc, b_eff)

    return out2.reshape(B, R, R, Cout)


# speedup vs baseline: 1.1321x; 1.1321x over previous
import functools

import jax
import jax.numpy as jnp
from jax.experimental import pallas as pl
from jax.experimental.pallas import tpu as pltpu

_R = 64
_NEG = -1e30


def _proj_kernel(idx_ref, f_ref, pv_ref, n_ref, wf_ref, wx01_ref, wxc_ref,
                 b_ref, o_ref, zbuf, a0, a1, a2, a3, a4, a5, a6, a7,
                 idxs, sems, *, ncb, unroll, pb):
    b = pl.program_id(0)
    c = pl.program_id(1)
    nch = pl.num_programs(1)
    nb = pl.num_programs(0)
    bufs = (a0, a1, a2, a3, a4, a5, a6, a7)

    g = b * nch + c
    slot = jax.lax.rem(c, 2)

    @pl.when(c == 0)
    def _idx0():
        pltpu.make_async_copy(idx_ref.at[g, 0], idxs.at[0],
                              sems.at[0]).start()
        pltpu.make_async_copy(idx_ref.at[g + 1, 0], idxs.at[1],
                              sems.at[1]).start()

    @pl.when(jnp.logical_and(c >= 1, c < nch - 1))
    def _idx_next():
        pltpu.make_async_copy(idx_ref.at[g + 1, 0], idxs.at[1 - slot],
                              sems.at[1 - slot]).start()

    pltpu.make_async_copy(idx_ref.at[g, 0], idxs.at[slot],
                          sems.at[slot]).wait()

    @pl.when(c == 0)
    def _init():
        for buf in bufs:
            buf[0:2 * pb:2, :] = jnp.full((pb, 128), _NEG, jnp.float32)
            buf[1:2 * pb:2, :] = jnp.zeros((pb, 128), jnp.float32)

    norm = n_ref[...]
    xpyp = pv_ref[...][:, 2:4]
    z = (jax.lax.dot_general(f_ref[0], wf_ref[...], (((0,), (0,)), ((), ())),
                             preferred_element_type=jnp.float32)
         + jnp.dot(xpyp, wx01_ref[...],
                   preferred_element_type=jnp.float32)
         + jnp.dot(norm, wxc_ref[...],
                   preferred_element_type=jnp.float32)
         + b_ref[...])
    zbuf[0:2 * ncb:2, :] = z
    zbuf[1:2 * ncb:2, 0:4] = jnp.concatenate(
        [norm, jnp.ones((ncb, 1), jnp.float32)], axis=1)

    mask2 = jax.lax.broadcasted_iota(jnp.int32, (2, 128), 0) == 0

    def body(j, carry):
        k0 = j * unroll
        for g in range(unroll // 8):
            ks = [k0 + 8 * g + t for t in range(8)]
            i2s = [pl.multiple_of(idxs[slot, k], 2) for k in ks]
            zns = [zbuf[pl.ds(2 * k, 2), :] for k in ks]
            olds = [bufs[t][pl.ds(i2s[t], 2), :] for t in range(8)]
            news = [jnp.where(mask2, jnp.maximum(olds[t], zns[t]),
                              olds[t] + zns[t]) for t in range(8)]
            for t in range(8):
                bufs[t][pl.ds(i2s[t], 2), :] = news[t]
        return carry

    jax.lax.fori_loop(0, ncb // unroll, body, 0)

    @pl.when(c == nch - 1)
    def _fin():
        ms = [buf[0:2 * pb:2, :] for buf in bufs]
        while len(ms) > 1:
            ms = [jnp.maximum(ms[i], ms[i + 1]) for i in range(0, len(ms), 2)]
        m = ms[0]
        ss = [buf[1:2 * pb:2, 0:4] for buf in bufs]
        while len(ss) > 1:
            ss = [ss[i] + ss[i + 1] for i in range(0, len(ss), 2)]
        s = ss[0]
        cnt = jnp.maximum(s[:, 3:4], 1.0)
        mean = s[:, 0:3] / cnt
        corr = jnp.dot(mean, wxc_ref[...],
                       preferred_element_type=jnp.float32)
        o_ref[...] = jnp.maximum(m - corr, 0.0)


def kernel(features, norm_coords, coords_int, p_v_dist, wf, wx, b_eff):
    B, C, Np = features.shape
    N = B * Np
    Cout = wf.shape[1]
    R = _R
    PB = R * R

    wx01 = wx[0:2]
    wxc = wx[2:5]
    li2 = ((coords_int[:, 2] * R + coords_int[:, 3]) * 2).astype(
        jnp.int32)

    NCB = 2048
    while Np % NCB:
        NCB //= 2
    nch = Np // NCB
    UNROLL = 32

    kb = functools.partial(_proj_kernel, ncb=NCB, unroll=UNROLL, pb=PB)
    out2 = pl.pallas_call(
        kb,
        out_shape=jax.ShapeDtypeStruct((B * PB, Cout), jnp.float32),
        grid=(B, nch),
        in_specs=[
            pl.BlockSpec(memory_space=pltpu.VMEM),
            pl.BlockSpec((1, C, NCB), lambda b, c: (b, 0, c)),
            pl.BlockSpec((NCB, 4), lambda b, c: (b * (Np // NCB) + c, 0)),
            pl.BlockSpec((NCB, 3), lambda b, c: (b * (Np // NCB) + c, 0)),
            pl.BlockSpec((C, Cout), lambda b, c: (0, 0)),
            pl.BlockSpec((2, Cout), lambda b, c: (0, 0)),
            pl.BlockSpec((3, Cout), lambda b, c: (0, 0)),
            pl.BlockSpec((1, Cout), lambda b, c: (0, 0)),
        ],
        out_specs=pl.BlockSpec((PB, Cout), lambda b, c: (b, 0)),
        scratch_shapes=[pltpu.VMEM((2 * NCB, Cout), jnp.float32)] +
                       [pltpu.VMEM((2 * PB, Cout), jnp.float32)
                        for _ in range(8)] +
                       [pltpu.SMEM((2, NCB), jnp.int32),
                        pltpu.SemaphoreType.DMA((2,))],
        compiler_params=pltpu.CompilerParams(
            dimension_semantics=("parallel", "arbitrary"),
            vmem_limit_bytes=48 * 1024 * 1024,
        ),
    )(li2.reshape(B * nch, 1, NCB), features, p_v_dist, norm_coords, wf, wx01, wxc, b_eff)

    return out2.reshape(B, R, R, Cout)


# speedup vs baseline: 1.1420x; 1.0087x over previous
import functools

import jax
import jax.numpy as jnp
from jax.experimental import pallas as pl
from jax.experimental.pallas import tpu as pltpu

_R = 64
_NEG = -1e30


def _proj_kernel(idx_ref, f_ref, pv_ref, n_ref, wf_ref, wx01_ref, wxc_ref,
                 b_ref, o_ref, zbuf, a0, a1, a2, a3, a4, a5, a6, a7,
                 idxs, sems, *, ncb, unroll, pb):
    b = pl.program_id(0)
    c = pl.program_id(1)
    nch = pl.num_programs(1)
    nb = pl.num_programs(0)
    bufs = (a0, a1, a2, a3, a4, a5, a6, a7)

    g = b * nch + c
    slot = jax.lax.rem(c, 2)

    @pl.when(c == 0)
    def _idx0():
        pltpu.make_async_copy(idx_ref.at[g, 0], idxs.at[0],
                              sems.at[0]).start()
        pltpu.make_async_copy(idx_ref.at[g + 1, 0], idxs.at[1],
                              sems.at[1]).start()

    @pl.when(jnp.logical_and(c >= 1, c < nch - 1))
    def _idx_next():
        pltpu.make_async_copy(idx_ref.at[g + 1, 0], idxs.at[1 - slot],
                              sems.at[1 - slot]).start()

    pltpu.make_async_copy(idx_ref.at[g, 0], idxs.at[slot],
                          sems.at[slot]).wait()

    @pl.when(c == 0)
    def _init():
        for buf in bufs:
            buf[0:2 * pb:2, :] = jnp.full((pb, 128), _NEG, jnp.float32)
            buf[1:2 * pb:2, :] = jnp.zeros((pb, 128), jnp.float32)

    norm = n_ref[...]
    xpyp = pv_ref[...][:, 2:4]
    z = (jax.lax.dot_general(f_ref[0], wf_ref[...], (((0,), (0,)), ((), ())),
                             preferred_element_type=jnp.float32)
         + jnp.dot(xpyp, wx01_ref[...],
                   preferred_element_type=jnp.float32)
         + jnp.dot(norm, wxc_ref[...],
                   preferred_element_type=jnp.float32)
         + b_ref[...])
    zbuf[0:2 * ncb:2, :] = z
    zbuf[1:2 * ncb:2, 0:4] = jnp.concatenate(
        [norm, jnp.ones((ncb, 1), jnp.float32)], axis=1)

    mask2 = jax.lax.broadcasted_iota(jnp.int32, (2, 128), 0) == 0

    def body(j, carry):
        k0 = j * unroll
        for g in range(unroll // 8):
            ks = [k0 + 8 * g + t for t in range(8)]
            i2s = [pl.multiple_of(idxs[slot, k], 2) for k in ks]
            zns = [zbuf[pl.ds(2 * k, 2), :] for k in ks]
            olds = [bufs[t][pl.ds(i2s[t], 2), :] for t in range(8)]
            news = [jnp.where(mask2, jnp.maximum(olds[t], zns[t]),
                              olds[t] + zns[t]) for t in range(8)]
            for t in range(8):
                bufs[t][pl.ds(i2s[t], 2), :] = news[t]
        return carry

    jax.lax.fori_loop(0, ncb // unroll, body, 0)

    @pl.when(c == nch - 1)
    def _fin():
        ms = [buf[0:2 * pb:2, :] for buf in bufs]
        while len(ms) > 1:
            ms = [jnp.maximum(ms[i], ms[i + 1]) for i in range(0, len(ms), 2)]
        m = ms[0]
        ss = [buf[1:2 * pb:2, 0:4] for buf in bufs]
        while len(ss) > 1:
            ss = [ss[i] + ss[i + 1] for i in range(0, len(ss), 2)]
        s = ss[0]
        cnt = jnp.maximum(s[:, 3:4], 1.0)
        mean = s[:, 0:3] / cnt
        corr = jnp.dot(mean, wxc_ref[...],
                       preferred_element_type=jnp.float32)
        o_ref[...] = jnp.maximum(m - corr, 0.0)


def kernel(features, norm_coords, coords_int, p_v_dist, wf, wx, b_eff):
    B, C, Np = features.shape
    N = B * Np
    Cout = wf.shape[1]
    R = _R
    PB = R * R

    wx01 = wx[0:2]
    wxc = wx[2:5]
    li2 = ((coords_int[:, 2] * R + coords_int[:, 3]) * 2).astype(
        jnp.int32)

    NCB = 4096
    while Np % NCB:
        NCB //= 2
    nch = Np // NCB
    UNROLL = 32

    kb = functools.partial(_proj_kernel, ncb=NCB, unroll=UNROLL, pb=PB)
    out2 = pl.pallas_call(
        kb,
        out_shape=jax.ShapeDtypeStruct((B * PB, Cout), jnp.float32),
        grid=(B, nch),
        in_specs=[
            pl.BlockSpec(memory_space=pltpu.VMEM),
            pl.BlockSpec((1, C, NCB), lambda b, c: (b, 0, c)),
            pl.BlockSpec((NCB, 4), lambda b, c: (b * (Np // NCB) + c, 0)),
            pl.BlockSpec((NCB, 3), lambda b, c: (b * (Np // NCB) + c, 0)),
            pl.BlockSpec((C, Cout), lambda b, c: (0, 0)),
            pl.BlockSpec((2, Cout), lambda b, c: (0, 0)),
            pl.BlockSpec((3, Cout), lambda b, c: (0, 0)),
            pl.BlockSpec((1, Cout), lambda b, c: (0, 0)),
        ],
        out_specs=pl.BlockSpec((PB, Cout), lambda b, c: (b, 0)),
        scratch_shapes=[pltpu.VMEM((2 * NCB, Cout), jnp.float32)] +
                       [pltpu.VMEM((2 * PB, Cout), jnp.float32)
                        for _ in range(8)] +
                       [pltpu.SMEM((2, NCB), jnp.int32),
                        pltpu.SemaphoreType.DMA((2,))],
        compiler_params=pltpu.CompilerParams(
            dimension_semantics=("parallel", "arbitrary"),
            vmem_limit_bytes=56 * 1024 * 1024,
        ),
    )(li2.reshape(B * nch, 1, NCB), features, p_v_dist, norm_coords, wf, wx01, wxc, b_eff)

    return out2.reshape(B, R, R, Cout)


# speedup vs baseline: 1.1427x; 1.0006x over previous
import functools

import jax
import jax.numpy as jnp
from jax.experimental import pallas as pl
from jax.experimental.pallas import tpu as pltpu

_R = 64
_NEG = -1e30


def _proj_kernel(idx_ref, f_ref, pv_ref, n_ref, wf_ref, wx01_ref, wxc_ref,
                 b_ref, o_ref, zbuf, a0, a1, a2, a3, a4, a5, a6, a7,
                 idxs, sems, *, ncb, unroll, pb, nchs):
    b = pl.program_id(0)
    c = pl.program_id(1)
    nch = pl.num_programs(1)
    nb = pl.num_programs(0)
    bufs = (a0, a1, a2, a3, a4, a5, a6, a7)

    g = b * nch + c
    slot = jax.lax.rem(c, 2)

    @pl.when(c == 0)
    def _idx0():
        pltpu.make_async_copy(idx_ref.at[g, 0], idxs.at[0],
                              sems.at[0]).start()
        if nchs > 1:
            pltpu.make_async_copy(idx_ref.at[g + 1, 0], idxs.at[1],
                                  sems.at[1]).start()

    if nchs > 2:
        @pl.when(jnp.logical_and(c >= 1, c < nch - 1))
        def _idx_next():
            pltpu.make_async_copy(idx_ref.at[g + 1, 0], idxs.at[1 - slot],
                                  sems.at[1 - slot]).start()

    pltpu.make_async_copy(idx_ref.at[g, 0], idxs.at[slot],
                          sems.at[slot]).wait()

    @pl.when(c == 0)
    def _init():
        for buf in bufs:
            buf[0:2 * pb:2, :] = jnp.full((pb, 128), _NEG, jnp.float32)
            buf[1:2 * pb:2, :] = jnp.zeros((pb, 128), jnp.float32)

    norm = n_ref[...]
    xpyp = pv_ref[...][:, 2:4]
    z = (jax.lax.dot_general(f_ref[0], wf_ref[...], (((0,), (0,)), ((), ())),
                             preferred_element_type=jnp.float32)
         + jnp.dot(xpyp, wx01_ref[...],
                   preferred_element_type=jnp.float32)
         + jnp.dot(norm, wxc_ref[...],
                   preferred_element_type=jnp.float32)
         + b_ref[...])
    zbuf[0:2 * ncb:2, :] = z
    zbuf[1:2 * ncb:2, 0:4] = jnp.concatenate(
        [norm, jnp.ones((ncb, 1), jnp.float32)], axis=1)

    mask2 = jax.lax.broadcasted_iota(jnp.int32, (2, 128), 0) == 0

    def body(j, carry):
        k0 = j * unroll
        for grp in range(unroll // 8):
            ks = [k0 + 8 * grp + t for t in range(8)]
            i2s = [pl.multiple_of(idxs[slot, k], 2) for k in ks]
            zns = [zbuf[pl.ds(2 * k, 2), :] for k in ks]
            olds = [bufs[t][pl.ds(i2s[t], 2), :] for t in range(8)]
            news = [jnp.where(mask2, jnp.maximum(olds[t], zns[t]),
                              olds[t] + zns[t]) for t in range(8)]
            for t in range(8):
                bufs[t][pl.ds(i2s[t], 2), :] = news[t]
        return carry

    jax.lax.fori_loop(0, ncb // unroll, body, 0)

    @pl.when(c == nch - 1)
    def _fin():
        ms = [buf[0:2 * pb:2, :] for buf in bufs]
        while len(ms) > 1:
            ms = [jnp.maximum(ms[i], ms[i + 1]) for i in range(0, len(ms), 2)]
        m = ms[0]
        ss = [buf[1:2 * pb:2, 0:4] for buf in bufs]
        while len(ss) > 1:
            ss = [ss[i] + ss[i + 1] for i in range(0, len(ss), 2)]
        s = ss[0]
        cnt = jnp.maximum(s[:, 3:4], 1.0)
        mean = s[:, 0:3] / cnt
        corr = jnp.dot(mean, wxc_ref[...],
                       preferred_element_type=jnp.float32)
        o_ref[...] = jnp.maximum(m - corr, 0.0)


def kernel(features, norm_coords, coords_int, p_v_dist, wf, wx, b_eff):
    B, C, Np = features.shape
    N = B * Np
    Cout = wf.shape[1]
    R = _R
    PB = R * R

    wx01 = wx[0:2]
    wxc = wx[2:5]
    li2 = ((coords_int[:, 2] * R + coords_int[:, 3]) * 2).astype(
        jnp.int32)

    NCB = 4096
    while Np % NCB:
        NCB //= 2
    nch = Np // NCB
    UNROLL = 32

    kb = functools.partial(_proj_kernel, ncb=NCB, unroll=UNROLL, pb=PB,
                           nchs=nch)
    out2 = pl.pallas_call(
        kb,
        out_shape=jax.ShapeDtypeStruct((B * PB, Cout), jnp.float32),
        grid=(B, nch),
        in_specs=[
            pl.BlockSpec(memory_space=pltpu.VMEM),
            pl.BlockSpec((1, C, NCB), lambda b, c: (b, 0, c)),
            pl.BlockSpec((NCB, 4), lambda b, c: (b * (Np // NCB) + c, 0)),
            pl.BlockSpec((NCB, 3), lambda b, c: (b * (Np // NCB) + c, 0)),
            pl.BlockSpec((C, Cout), lambda b, c: (0, 0)),
            pl.BlockSpec((2, Cout), lambda b, c: (0, 0)),
            pl.BlockSpec((3, Cout), lambda b, c: (0, 0)),
            pl.BlockSpec((1, Cout), lambda b, c: (0, 0)),
        ],
        out_specs=pl.BlockSpec((PB, Cout), lambda b, c: (b, 0)),
        scratch_shapes=[pltpu.VMEM((2 * NCB, Cout), jnp.float32)] +
                       [pltpu.VMEM((2 * PB, Cout), jnp.float32)
                        for _ in range(8)] +
                       [pltpu.SMEM((2, NCB), jnp.int32),
                        pltpu.SemaphoreType.DMA((2,))],
        compiler_params=pltpu.CompilerParams(
            dimension_semantics=("parallel", "arbitrary"),
            vmem_limit_bytes=56 * 1024 * 1024,
        ),
    )(li2.reshape(B * nch, 1, NCB), features, p_v_dist, norm_coords, wf, wx01, wxc, b_eff)

    return out2.reshape(B, R, R, Cout)
